# baseline (speedup 1.0000x reference)
import functools

import jax
import jax.numpy as jnp
from jax.experimental import pallas as pl
from jax.experimental.pallas import tpu as pltpu


_EPS = 1e-5


def _dot_t(a, w):
    return jax.lax.dot_general(
        a.astype(jnp.bfloat16), w.astype(jnp.bfloat16),
        (((1,), (1,)), ((), ())), preferred_element_type=jnp.float32
    )


def _bn_affine(st, n_rows, g, be):
    s = st[0:1, :]
    q = st[1:2, :]
    m = s * (1.0 / n_rows)
    v = q * (1.0 / n_rows) - m * m
    a = g * jax.lax.rsqrt(v + _EPS)
    o = be - m * a
    return a, o


def _lrelu(t):
    return jnp.maximum(t, 0.1 * t)


def _col_stats(y):
    s = jnp.sum(y, axis=0, keepdims=True)
    q = jnp.sum(y * y, axis=0, keepdims=True)
    return jnp.concatenate([s, q], axis=0)


def _fused_kernel(x_ref, w1_ref, w2_ref, w3_ref, w4_ref, pars_ref, b4_ref,
                  out_ref, act_ref, st_ref, *, n_rows, bn):
    p = pl.program_id(0)
    i = pl.program_id(1)
    rows = pl.ds(i * bn, bn)

    @pl.when(jnp.logical_and(p == 0, i == 0))
    def _():
        st_ref[...] = jnp.zeros_like(st_ref)

    @pl.when(p == 0)
    def _():
        z1 = _dot_t(x_ref[...], w1_ref[...])
        st_ref[0:2, :] += _col_stats(z1)
        act_ref[rows, :] = z1.astype(jnp.bfloat16)

    @pl.when(p == 1)
    def _():
        x = x_ref[...]
        a1, o1 = _bn_affine(st_ref[0:2, :], n_rows,
                            pars_ref[0:1, :], pars_ref[1:2, :])
        t = act_ref[rows, :].astype(jnp.float32) * a1 + o1
        x1 = _lrelu(t) + 0.1 * x
        x1b = x1.astype(jnp.bfloat16)
        act_ref[rows, :] = x1b
        z2 = _dot_t(x1b, w2_ref[...])
        st_ref[2:4, :] += _col_stats(z2)

    @pl.when(p == 2)
    def _():
        x1b = act_ref[rows, :]
        a2, o2 = _bn_affine(st_ref[2:4, :], n_rows,
                            pars_ref[2:3, :], pars_ref[3:4, :])
        t = _dot_t(x1b, w2_ref[...]) * a2 + o2
        x2 = _lrelu(t) + 0.1 * x1b.astype(jnp.float32)
        x2b = x2.astype(jnp.bfloat16)
        act_ref[rows, :] = x2b
        z3 = _dot_t(x2b, w3_ref[...])
        st_ref[4:6, :] += _col_stats(z3)

    @pl.when(p == 3)
    def _():
        x2b = act_ref[rows, :]
        a3, o3 = _bn_affine(st_ref[4:6, :], n_rows,
                            pars_ref[4:5, :], pars_ref[5:6, :])
        t = _dot_t(x2b, w3_ref[...]) * a3 + o3
        x3 = _lrelu(t) + 0.1 * x2b.astype(jnp.float32)
        o8 = jax.lax.dot_general(
            w4_ref[...].astype(jnp.bfloat16), x3.astype(jnp.bfloat16),
            (((1,), (1,)), ((), ())), preferred_element_type=jnp.float32)
        out_ref[...] = (o8[0:1, :] + b4_ref[0, 0]).reshape(out_ref.shape)


def _pick_block(n):
    for bn in (25000, 20000, 10000, 4000, 2000, 1000, 800, 500, 250, 200, 104, 100, 50, 40,
               25, 20, 8):
        if n % bn == 0 and bn % 8 == 0:
            return bn
    return n


def kernel(x, W1, b1, g1, be1, W2, b2, g2, be2, W3, b3, g3, be3, W4, b4):
    n, d = x.shape
    bn = _pick_block(n)
    nb = n // bn

    row = lambda v: v.reshape(1, d)
    pars = jnp.concatenate(
        [row(g1), row(be1), row(g2), row(be2), row(g3), row(be3)], axis=0)
    w4p = jnp.concatenate([W4, jnp.zeros((7, d), jnp.float32)], axis=0)

    xs = pl.BlockSpec((bn, d), lambda p, i: (jnp.where(p < 2, i, 0), 0))
    ws = pl.BlockSpec((d, d), lambda p, i: (0, 0))

    out_t = pl.pallas_call(
        functools.partial(_fused_kernel, n_rows=float(n), bn=bn),
        grid=(4, nb),
        in_specs=[
            xs, ws, ws, ws,
            pl.BlockSpec((8, d), lambda p, i: (0, 0)),
            pl.BlockSpec((6, d), lambda p, i: (0, 0)),
            pl.BlockSpec((1, 1), lambda p, i: (0, 0)),
        ],
        out_specs=pl.BlockSpec((1, 1, bn),
                               lambda p, i: (jnp.where(p == 3, i, 0), 0, 0)),
        out_shape=jax.ShapeDtypeStruct((nb, 1, bn), jnp.float32),
        scratch_shapes=[
            pltpu.VMEM((n, d), jnp.bfloat16),
            pltpu.VMEM((8, d), jnp.float32),
        ],
        compiler_params=pltpu.CompilerParams(
            dimension_semantics=("arbitrary", "arbitrary"),
            vmem_limit_bytes=100 * 1024 * 1024,
        ),
    )(x, W1, W2, W3, w4p, pars, b4.reshape(1, 1))

    return out_t.reshape(n, 1)


# speedup vs baseline: 1.3098x; 1.3098x over previous
import functools

import jax
import jax.numpy as jnp
from jax.experimental import pallas as pl
from jax.experimental.pallas import tpu as pltpu


_EPS = 1e-5


def _dot_t(a, w):
    return jax.lax.dot_general(
        a.astype(jnp.bfloat16), w.astype(jnp.bfloat16),
        (((1,), (1,)), ((), ())), preferred_element_type=jnp.float32
    )


def _bn_affine(st, n_rows, g, be):
    s = st[0:1, :]
    q = st[1:2, :]
    m = s * (1.0 / n_rows)
    v = q * (1.0 / n_rows) - m * m
    a = g * jax.lax.rsqrt(v + _EPS)
    o = be - m * a
    return a, o


def _lrelu(t):
    return jnp.maximum(t, 0.1 * t)


def _col_stats(y):
    s = jnp.sum(y, axis=0, keepdims=True)
    q = jnp.sum(y * y, axis=0, keepdims=True)
    return jnp.concatenate([s, q], axis=0)


def _fused_kernel(x_ref, w1_ref, w2_ref, w3_ref, w4_ref, pars_ref, b4_ref,
                  out_ref, act_ref, st_ref, *, n_rows, bn):
    p = pl.program_id(0)
    i = pl.program_id(1)
    rows = pl.ds(i * bn, bn)

    @pl.when(jnp.logical_and(p == 0, i == 0))
    def _():
        st_ref[...] = jnp.zeros_like(st_ref)

    @pl.when(p == 0)
    def _():
        z1 = _dot_t(x_ref[...], w1_ref[...])
        st_ref[0:2, :] += _col_stats(z1)
        act_ref[rows, :] = z1.astype(jnp.bfloat16)

    @pl.when(p == 1)
    def _():
        x = x_ref[...]
        a1, o1 = _bn_affine(st_ref[0:2, :], n_rows,
                            pars_ref[0:1, :], pars_ref[1:2, :])
        t = act_ref[rows, :].astype(jnp.float32) * a1 + o1
        x1 = _lrelu(t) + 0.1 * x
        x1b = x1.astype(jnp.bfloat16)
        act_ref[rows, :] = x1b
        z2 = _dot_t(x1b, w2_ref[...])
        st_ref[2:4, :] += _col_stats(z2)

    @pl.when(p == 2)
    def _():
        x1b = act_ref[rows, :]
        a2, o2 = _bn_affine(st_ref[2:4, :], n_rows,
                            pars_ref[2:3, :], pars_ref[3:4, :])
        t = _dot_t(x1b, w2_ref[...]) * a2 + o2
        x2 = _lrelu(t) + 0.1 * x1b.astype(jnp.float32)
        x2b = x2.astype(jnp.bfloat16)
        act_ref[rows, :] = x2b
        z3 = _dot_t(x2b, w3_ref[...])
        st_ref[4:6, :] += _col_stats(z3)

    @pl.when(p == 3)
    def _():
        x2b = act_ref[rows, :]
        a3, o3 = _bn_affine(st_ref[4:6, :], n_rows,
                            pars_ref[4:5, :], pars_ref[5:6, :])
        t = _dot_t(x2b, w3_ref[...]) * a3 + o3
        x3 = _lrelu(t) + 0.1 * x2b.astype(jnp.float32)
        o8 = jax.lax.dot_general(
            w4_ref[...].astype(jnp.bfloat16), x3.astype(jnp.bfloat16),
            (((1,), (1,)), ((), ())), preferred_element_type=jnp.float32)
        out_ref[...] = (o8[0:1, :] + b4_ref[0, 0]).reshape(out_ref.shape)


def _pick_block(n):
    for bn in (20000, 10000, 4000, 2000, 1000, 800, 500, 250, 200, 104, 100, 50, 40,
               25, 20, 8):
        if n % bn == 0 and bn % 8 == 0:
            return bn
    return n


def kernel(x, W1, b1, g1, be1, W2, b2, g2, be2, W3, b3, g3, be3, W4, b4):
    n, d = x.shape
    bn = _pick_block(n)
    nb = n // bn

    row = lambda v: v.reshape(1, d)
    pars = jnp.concatenate(
        [row(g1), row(be1), row(g2), row(be2), row(g3), row(be3)], axis=0)
    w4p = jnp.concatenate([W4, jnp.zeros((7, d), jnp.float32)], axis=0)

    xs = pl.BlockSpec((bn, d), lambda p, i: (jnp.where(p < 2, i, 0), 0))
    ws = pl.BlockSpec((d, d), lambda p, i: (0, 0))

    out_t = pl.pallas_call(
        functools.partial(_fused_kernel, n_rows=float(n), bn=bn),
        grid=(4, nb),
        in_specs=[
            xs, ws, ws, ws,
            pl.BlockSpec((8, d), lambda p, i: (0, 0)),
            pl.BlockSpec((6, d), lambda p, i: (0, 0)),
            pl.BlockSpec((1, 1), lambda p, i: (0, 0)),
        ],
        out_specs=pl.BlockSpec((1, 1, bn),
                               lambda p, i: (jnp.where(p == 3, i, 0), 0, 0)),
        out_shape=jax.ShapeDtypeStruct((nb, 1, bn), jnp.float32),
        scratch_shapes=[
            pltpu.VMEM((n, d), jnp.bfloat16),
            pltpu.VMEM((8, d), jnp.float32),
        ],
        compiler_params=pltpu.CompilerParams(
            dimension_semantics=("arbitrary", "arbitrary"),
            vmem_limit_bytes=100 * 1024 * 1024,
        ),
    )(x, W1, W2, W3, w4p, pars, b4.reshape(1, 1))

    return out_t.reshape(n, 1)
